# Initial kernel scaffold; baseline (speedup 1.0000x reference)
import jax
import jax.numpy as jnp
from jax.experimental import pallas as pl


def kernel(logit, tgt, tgt_lens):
    raise NotImplementedError("write your pallas kernel here")


# speedup vs baseline: 11.3606x; 11.3606x over previous
import jax
import jax.numpy as jnp
from jax import lax
from jax.experimental import pallas as pl
from jax.experimental.pallas import tpu as pltpu

_B, _S, _V = 16, 128, 5000
_N = 4
_KEY_HI, _KEY_LO = 0, 42


def _threefry2x32(x0, x1):
    k0 = jnp.uint32(_KEY_HI)
    k1 = jnp.uint32(_KEY_LO)
    ks2 = jnp.uint32(0x1BD11BDA) ^ k0 ^ k1
    ks = (k0, k1, ks2)
    rots = ((13, 15, 26, 6), (17, 29, 16, 24))
    x0 = x0 + k0
    x1 = x1 + k1
    for i in range(5):
        for r in rots[i % 2]:
            x0 = x0 + x1
            x1 = (x1 << r) | (x1 >> (32 - r))
            x1 = x1 ^ x0
        x0 = x0 + ks[(i + 1) % 3]
        x1 = x1 + ks[(i + 2) % 3] + jnp.uint32(i + 1)
    return x0, x1


def _mwer_body(len_ref, logit_ref, tgt_ref, out_ref):
    b = pl.program_id(0)
    x = logit_ref[0]
    iota_v = lax.broadcasted_iota(jnp.int32, (_S, _V), 1)

    m1 = jnp.max(x, axis=-1, keepdims=True)
    i1 = jnp.min(jnp.where(x == m1, iota_v, _V), axis=-1, keepdims=True)
    x2 = jnp.where(iota_v == i1, -jnp.inf, x)
    m2 = jnp.max(x2, axis=-1, keepdims=True)
    i2 = jnp.min(jnp.where(x2 == m2, iota_v, _V), axis=-1, keepdims=True)

    n_iota = lax.broadcasted_iota(jnp.int32, (_N, _S), 0)
    s_iota = lax.broadcasted_iota(jnp.int32, (_N, _S), 1)
    i1r = jnp.broadcast_to(i1.reshape(1, _S), (_N, _S))
    flat = ((n_iota * _B + b) * _S + s_iota) * _V + i1r
    o0, o1 = _threefry2x32(jnp.zeros((_N, _S), jnp.uint32),
                           flat.astype(jnp.uint32))
    bits = o0 ^ o1
    masked = (bits >> 31) == 0

    pad = s_iota >= len_ref[b]
    v1 = jnp.broadcast_to(m1.reshape(1, _S), (_N, _S))
    v2 = jnp.broadcast_to(m2.reshape(1, _S), (_N, _S))
    i2r = jnp.broadcast_to(i2.reshape(1, _S), (_N, _S))

    sel_v = jnp.where(masked, v2, v1)
    sel_v = jnp.where(pad, 0.0, sel_v)
    a = jnp.sum(sel_v, axis=-1, keepdims=True)

    pred = jnp.where(masked, i2r, i1r)
    tgt = jnp.broadcast_to(tgt_ref[0, 0].reshape(1, _S), (_N, _S))
    err = jnp.sum(
        jnp.where(pad, 0.0, (tgt != pred).astype(jnp.float32)),
        axis=-1, keepdims=True)

    md = jnp.max(a, axis=0, keepdims=True)
    w = jnp.exp(a - md)
    normal = w / jnp.sum(w, axis=0, keepdims=True)
    dev = err - jnp.mean(err, axis=0, keepdims=True)
    out_ref[0] = jnp.sum(normal * dev, axis=0, keepdims=True)


def kernel(logit, tgt, tgt_lens):
    tgt3 = tgt.reshape(_B, 1, _S)
    loss = pl.pallas_call(
        _mwer_body,
        grid=(_B,),
        in_specs=[
            pl.BlockSpec(memory_space=pltpu.SMEM),
            pl.BlockSpec((1, _S, _V), lambda b: (b, 0, 0)),
            pl.BlockSpec((1, 1, _S), lambda b: (b, 0, 0)),
        ],
        out_specs=pl.BlockSpec((1, 1, 1), lambda b: (b, 0, 0)),
        out_shape=jax.ShapeDtypeStruct((_B, 1, 1), jnp.float32),
    )(tgt_lens, logit, tgt3)
    return jnp.mean(loss)
